# baseline (speedup 1.0000x reference)
import jax
import jax.numpy as jnp
from jax import lax
from jax.experimental import pallas as pl
from jax.experimental.pallas import tpu as pltpu
from jax.experimental.pallas import tpu_sc as plsc

_JITTER_EPS = 0.1
_NEG_INF = float("-inf")
_E = 64
_N_TOKENS = 16384
_N_WORKERS = 32
_ROWS_PER_W = _N_TOKENS // _N_WORKERS


def _tc_scores_kernel(*refs):
    *x_refs, w_ref, scores_ref, gates_ref = refs
    n_split = len(x_refs)
    w = w_ref[...]
    ks = w.shape[1] // n_split
    scores = None
    for j, x_ref in enumerate(x_refs):
        part = lax.dot_general(
            x_ref[...],
            w[:, j * ks : (j + 1) * ks],
            (((1,), (1,)), ((), ())),
            preferred_element_type=jnp.float32,
        )
        scores = part if scores is None else scores + part
    scores_ref[...] = scores
    max_logit = jnp.max(scores, axis=-1, keepdims=True)
    ex0 = jnp.exp(scores - max_logit)
    gates_ref[...] = ex0 / jnp.sum(ex0, axis=-1, keepdims=True)


def _sc_routing(scores_hbm, mult_hbm, sel_hbm, sbuf, mult_buf, sel_buf):
    c = lax.axis_index("c")
    s = lax.axis_index("s")
    wid = s * 2 + c
    base = wid * _ROWS_PER_W
    pltpu.sync_copy(scores_hbm.at[pl.ds(base, _ROWS_PER_W)], sbuf)

    iota16 = lax.iota(jnp.int32, 16)
    thr = 2.0 * _JITTER_EPS

    perms = [jnp.bitwise_xor(iota16, sh) for sh in (8, 4, 2, 1)]

    def _butterfly(vec, op):
        for p in perms:
            vec = op(vec, vec.at[p].get(mode="promise_in_bounds"))
        return vec

    def one_row(r):
        v = [sbuf[r, pl.ds(16 * j, 16)] for j in range(4)]

        def reduce_max4(ws):
            return _butterfly(
                jnp.maximum(jnp.maximum(ws[0], ws[1]), jnp.maximum(ws[2], ws[3])),
                jnp.maximum,
            )

        def first_index_where(eqs):
            idx = [jnp.where(eqs[j], iota16 + 16 * j, _E) for j in range(4)]
            return _butterfly(
                jnp.minimum(jnp.minimum(idx[0], idx[1]), jnp.minimum(idx[2], idx[3])),
                jnp.minimum,
            )

        m = reduce_max4(v)
        max_ind = first_index_where([v[j] == m for j in range(4)])
        ex = [jnp.exp(v[j] - m) for j in range(4)]
        masked = [(m - v[j]) > thr * jnp.maximum(jnp.abs(v[j]), m) for j in range(4)]
        ex1 = [jnp.where(masked[j], 0.0, ex[j]) for j in range(4)]
        sum1 = _butterfly(ex1[0] + ex1[1] + ex1[2] + ex1[3], jnp.add)
        inv1 = 1.0 / sum1
        mg_ind = first_index_where([ex1[j] / sum1 == inv1 for j in range(4)])
        mfo1 = jnp.where(max_ind == mg_ind, jnp.float32(1.0), jnp.float32(0.3333))
        mult1 = inv1 * mfo1

        is_sel = [iota16 + 16 * j == max_ind for j in range(4)]
        ms = [jnp.where(is_sel[j], _NEG_INF, v[j]) for j in range(4)]
        m2 = reduce_max4(ms)
        max_ind2 = first_index_where([ms[j] == m2 for j in range(4)])
        masked2 = [
            (m2 - v[j]) > thr * jnp.maximum(jnp.abs(v[j]), m2) for j in range(4)
        ]
        ex2 = [
            jnp.where(jnp.logical_or(masked2[j], is_sel[j]), 0.0, jnp.exp(v[j] - m2))
            for j in range(4)
        ]
        sum2 = _butterfly(ex2[0] + ex2[1] + ex2[2] + ex2[3], jnp.add)
        inv2 = 1.0 / sum2
        mg2_ind = first_index_where([ex2[j] / sum2 == inv2 for j in range(4)])
        mfo2 = jnp.where(max_ind2 == mg2_ind, jnp.float32(1.0), jnp.float32(0.3333))
        mult2 = inv2 * mfo2
        return mult1, mult2, max_ind, max_ind2

    def blk_body(b, carry):
        mvec = jnp.zeros((16,), jnp.float32)
        svec = jnp.zeros((16,), jnp.int32)
        for rr in range(8):
            mult1, mult2, i1, i2 = one_row(b * 8 + rr)
            mvec = jnp.where(iota16 == 2 * rr, mult1, mvec)
            mvec = jnp.where(iota16 == 2 * rr + 1, mult2, mvec)
            svec = jnp.where(iota16 == 2 * rr, i1, svec)
            svec = jnp.where(iota16 == 2 * rr + 1, i2, svec)
        mult_buf[pl.ds(b * 16, 16)] = mvec
        sel_buf[pl.ds(b * 16, 16)] = svec
        return carry

    lax.fori_loop(0, _ROWS_PER_W // 8, blk_body, 0)
    pltpu.sync_copy(mult_buf, mult_hbm.at[pl.ds(base * 2, _ROWS_PER_W * 2)])
    pltpu.sync_copy(sel_buf, sel_hbm.at[pl.ds(base * 2, _ROWS_PER_W * 2)])


def kernel(x, W):
    n_tokens, d_model = x.shape
    n_experts = W.shape[0]
    t_blk = 1024
    n_split = 4
    ks = d_model // n_split
    grid = (n_tokens // t_blk,)
    scores, gates = pl.pallas_call(
        _tc_scores_kernel,
        grid=grid,
        in_specs=[
            pl.BlockSpec((t_blk, ks), lambda i, _j=j: (i, _j))
            for j in range(n_split)
        ]
        + [
            pl.BlockSpec((n_experts, d_model), lambda i: (0, 0)),
        ],
        out_specs=[
            pl.BlockSpec((t_blk, n_experts), lambda i: (i, 0)),
            pl.BlockSpec((t_blk, n_experts), lambda i: (i, 0)),
        ],
        out_shape=[
            jax.ShapeDtypeStruct((n_tokens, n_experts), jnp.float32),
            jax.ShapeDtypeStruct((n_tokens, n_experts), jnp.float32),
        ],
    )(*([x] * n_split), W)

    mesh = plsc.VectorSubcoreMesh(core_axis_name="c", subcore_axis_name="s")
    mult_flat, sel_flat = pl.kernel(
        _sc_routing,
        out_type=[
            jax.ShapeDtypeStruct((n_tokens * 2,), jnp.float32),
            jax.ShapeDtypeStruct((n_tokens * 2,), jnp.int32),
        ],
        mesh=mesh,
        scratch_types=[
            pltpu.VMEM((_ROWS_PER_W, _E), jnp.float32),
            pltpu.VMEM((_ROWS_PER_W * 2,), jnp.float32),
            pltpu.VMEM((_ROWS_PER_W * 2,), jnp.int32),
        ],
    )(scores)
    return mult_flat.reshape(n_tokens, 2), gates, sel_flat.reshape(n_tokens, 2)


# speedup vs baseline: 1.5513x; 1.5513x over previous
import jax
import jax.numpy as jnp
from jax import lax
from jax.experimental import pallas as pl

_JITTER_EPS = 0.1
_NEG_INF = float("-inf")


def _router_kernel(*refs):
    *x_refs, w_ref, gates_ref, mult_ref, sel_ref = refs
    n_split = len(x_refs)
    w = w_ref[...]
    ks = w.shape[1] // n_split
    scores = None
    for j, x_ref in enumerate(x_refs):
        part = lax.dot_general(
            x_ref[...],
            w[:, j * ks : (j + 1) * ks],
            (((1,), (1,)), ((), ())),
            preferred_element_type=jnp.float32,
        )
        scores = part if scores is None else scores + part

    t, e = scores.shape
    iota = lax.broadcasted_iota(jnp.int32, (t, e), 1)
    thr = 2.0 * _JITTER_EPS

    def argmin_at(eq_tile):
        return jnp.min(jnp.where(eq_tile, iota, e), axis=-1, keepdims=True)

    max_logit = jnp.max(scores, axis=-1, keepdims=True)
    max_ind = argmin_at(scores == max_logit)
    ex0 = jnp.exp(scores - max_logit)
    sum0 = jnp.sum(ex0, axis=-1, keepdims=True)
    gates_ref[...] = ex0 / sum0

    factor = jnp.maximum(jnp.abs(scores), max_logit)
    mask = (max_logit - scores) > thr * factor
    ex1 = jnp.where(mask, 0.0, ex0)
    sum1 = jnp.sum(ex1, axis=-1, keepdims=True)
    inv1 = 1.0 / sum1
    mg_max_ind = argmin_at(ex1 == 1.0)
    mask_for_one = 0.3333 + 0.6667 * (max_ind == mg_max_ind).astype(jnp.float32)
    mult1 = inv1 * mask_for_one

    is_sel = iota == max_ind
    ms = jnp.where(is_sel, _NEG_INF, scores)
    max_logit2 = jnp.max(ms, axis=-1, keepdims=True)
    max_ind2 = argmin_at(ms == max_logit2)
    factor2 = jnp.maximum(jnp.abs(scores), max_logit2)
    mask2 = (max_logit2 - scores) > thr * factor2
    ex2 = jnp.where(jnp.logical_or(mask2, is_sel), 0.0, jnp.exp(scores - max_logit2))
    sum2 = jnp.sum(ex2, axis=-1, keepdims=True)
    inv2 = 1.0 / sum2
    mg2_max_ind = argmin_at(ex2 == 1.0)
    mask_for_one2 = 0.3333 + 0.6667 * (max_ind2 == mg2_max_ind).astype(jnp.float32)
    mult2 = inv2 * mask_for_one2

    mult_ref[...] = jnp.concatenate([mult1, mult2], axis=-1)
    sel_ref[...] = jnp.concatenate([max_ind, max_ind2], axis=-1)


def kernel(x, W):
    n_tokens, d_model = x.shape
    n_experts = W.shape[0]
    t_blk = 1024
    n_split = 4
    ks = d_model // n_split
    grid = (n_tokens // t_blk,)
    gates, mult, sel = pl.pallas_call(
        _router_kernel,
        grid=grid,
        in_specs=[
            pl.BlockSpec((t_blk, ks), lambda i, _j=j: (i, _j))
            for j in range(n_split)
        ]
        + [
            pl.BlockSpec((n_experts, d_model), lambda i: (0, 0)),
        ],
        out_specs=[
            pl.BlockSpec((t_blk, n_experts), lambda i: (i, 0)),
            pl.BlockSpec((t_blk, 2), lambda i: (i, 0)),
            pl.BlockSpec((t_blk, 2), lambda i: (i, 0)),
        ],
        out_shape=[
            jax.ShapeDtypeStruct((n_tokens, n_experts), jnp.float32),
            jax.ShapeDtypeStruct((n_tokens, 2), jnp.float32),
            jax.ShapeDtypeStruct((n_tokens, 2), jnp.int32),
        ],
    )(*([x] * n_split), W)
    return mult, gates, sel
